# Initial kernel scaffold; baseline (speedup 1.0000x reference)
import jax
import jax.numpy as jnp
from jax.experimental import pallas as pl


def kernel(lg, rg, kpl, kpr, scores, Q):
    raise NotImplementedError("write your pallas kernel here")


# speedup vs baseline: 15.3207x; 15.3207x over previous
import functools

import jax
import jax.numpy as jnp
from jax import lax
from jax.experimental import pallas as pl
from jax.experimental.pallas import tpu as pltpu
from jax.experimental.pallas import tpu_sc as plsc

B, N, H, W = 8, 2000, 512, 512
NP = 2048
RB = 256
NRB = NP // RB
K = 5
NKP = B * NP
GRP = 16
NC, NS = 2, 16
NW = NC * NS
GPI = NKP // GRP
GPW = GPI // NW
KPW = GPW * GRP


def _smooth_l1(x, y, beta):
    d = jnp.abs(x - y)
    return jnp.where(d < beta, 0.5 * d * d / beta, d - 0.5 * beta)


def _sc_gather(imgs, klx, kly, krx, kry):
    mesh = plsc.VectorSubcoreMesh(core_axis_name="c", subcore_axis_name="s")

    @functools.partial(
        pl.kernel,
        out_type=[jax.ShapeDtypeStruct((GPI, 64 * GRP), jnp.float32),
                  jax.ShapeDtypeStruct((GPI, 64 * GRP), jnp.float32)],
        mesh=mesh,
        scratch_types=[
            pltpu.VMEM((KPW,), jnp.float32),
            pltpu.VMEM((KPW,), jnp.float32),
            pltpu.VMEM((64 * GRP,), jnp.int32),
            pltpu.VMEM((64 * GRP,), jnp.float32),
            pltpu.SemaphoreType.DMA,
        ],
    )
    def body(imgs_hbm, klx_h, kly_h, krx_h, kry_h, outl, outr,
             kxv, kyv, idxv, gv, sem):
        wid = lax.axis_index("s") * NC + lax.axis_index("c")
        lane = lax.iota(jnp.int32, GRP)

        def do_image(kx_h, ky_h, out, side):
            base_kp = wid * KPW
            pltpu.sync_copy(kx_h.at[pl.ds(base_kp, KPW)], kxv)
            pltpu.sync_copy(ky_h.at[pl.ds(base_kp, KPW)], kyv)

            def group(t, carry):
                off = t * GRP
                kx = kxv[pl.ds(off, GRP)]
                ky = kyv[pl.ds(off, GRP)]
                kp = base_kp + off + lane
                bimg = (kp >> 11) + (side * B)
                base = bimg << 18
                xi = kx.astype(jnp.int32)
                yt = ky.astype(jnp.int32)
                yi = yt - jnp.where(ky < yt.astype(jnp.float32), 1, 0)
                for i in range(8):
                    yr = jnp.minimum(jnp.maximum(yi + (i - 3), 0), H - 1)
                    rowb = base + (yr << 9)
                    for j in range(8):
                        xc = jnp.minimum(jnp.maximum(xi + (j - 3), 0), W - 1)
                        idxv[pl.ds((i * 8 + j) * GRP, GRP)] = rowb + xc
                pltpu.async_copy(imgs_hbm.at[idxv], gv, sem).wait()
                pltpu.sync_copy(gv, out.at[wid * GPW + t])
                return carry

            lax.fori_loop(0, GPW, group, 0)

        do_image(klx_h, kly_h, outl, 0)
        do_image(krx_h, kry_h, outr, 1)

    return body(imgs, klx, kly, krx, kry)


def _pinn_features(kx, ky, krx, sc, q):
    d = kx - krx

    def bf(v):
        return v.astype(jnp.bfloat16).astype(jnp.float32)

    kxb, kyb, db = bf(kx), bf(ky), bf(d)
    proj = []
    for j in range(4):
        proj.append(((kxb * bf(q[j][0]) + kyb * bf(q[j][1]))
                     + db * bf(q[j][2])) + bf(q[j][3]))
    wc = jnp.maximum(proj[3], 1e-6)
    x3 = proj[0] / wc
    y3 = proj[1] / wc
    z3 = proj[2] / wc
    mask = ((z3 > 100.0) & (z3 < 30000.0) & (sc > 0.1)).astype(jnp.float32)
    return x3 / 1000.0, z3 / 1000.0, y3 / 1000.0, mask


def _cdist_body(q_ref, kxc, kyc, krxc, scc, kxr, kyr, krxr, scr, out_ref):
    rb = pl.program_id(1)
    q = [[q_ref[0, j, k] for k in range(4)] for j in range(4)]

    xs_c, zs_c, hgt_c, mask_c = _pinn_features(
        kxc[...].reshape(1, NP), kyc[...].reshape(1, NP),
        krxc[...].reshape(1, NP), scc[...].reshape(1, NP), q)
    xs_r, zs_r, hgt_r, mask_r = _pinn_features(
        kxr[...], kyr[...], krxr[...], scr[...], q)

    dsq = (xs_r - xs_c) ** 2 + (zs_r - zs_c) ** 2
    dsq = jnp.maximum(dsq, 1e-24)
    big = jnp.float32(1e20)
    dmat = jnp.where(mask_c > 0.5, dsq, big)

    ii = lax.broadcasted_iota(jnp.int32, (RB, NP), 1)
    sqs, hs = [], []
    for r in range(K + 1):
        m = jnp.min(dmat, axis=1, keepdims=True)
        cand = jnp.where(dmat == m, ii, NP)
        jmin = jnp.min(cand, axis=1, keepdims=True)
        sel = cand == jmin
        h = jnp.sum(jnp.where(sel, hgt_c, 0.0), axis=1, keepdims=True)
        sqs.append(m)
        hs.append(h)
        if r < K:
            dmat = jnp.where(sel, big, dmat)

    local_mean = (hs[1] + hs[2] + hs[3] + hs[4] + hs[5]) * jnp.float32(0.2)
    sl1 = _smooth_l1(hgt_r, local_mean, jnp.float32(0.01))
    pen = jnp.zeros_like(hgt_r)
    for r in range(1, K + 1):
        ndist = jnp.maximum(jnp.sqrt(sqs[r]), 0.001)
        pen = pen + jnp.maximum(jnp.abs(hs[r] - hgt_r) / ndist - 0.4, 0.0)

    s_sm = jnp.sum(sl1 * mask_r)
    s_pen = jnp.sum(pen * mask_r)
    s_h = jnp.sum(hgt_r * mask_r)
    s_cnt = jnp.sum(mask_r)

    lane = lax.broadcasted_iota(jnp.int32, (1, 128), 1)
    vec = (jnp.where(lane == 0, s_sm, 0.0) + jnp.where(lane == 1, s_pen, 0.0)
           + jnp.where(lane == 2, s_h, 0.0) + jnp.where(lane == 3, s_cnt, 0.0))

    @pl.when(rb == 0)
    def _():
        out_ref[...] = jnp.zeros_like(out_ref)

    out_ref[...] += vec.reshape(1, 1, 128)


def _cdist_call(q, kx, ky, krx, sc, interpret=False):
    col = pl.BlockSpec((1, 1, NP), lambda b, rb: (b, 0, 0))
    row = pl.BlockSpec((RB, 1), lambda b, rb: (b * NRB + rb, 0))
    args = []
    for arr in (kx, ky, krx, sc):
        args.append(arr.reshape(B, 1, NP))
    for arr in (kx, ky, krx, sc):
        args.append(arr.reshape(B * NP, 1))
    out = pl.pallas_call(
        _cdist_body,
        grid=(B, NRB),
        in_specs=[
            pl.BlockSpec((1, 4, 4), lambda b, rb: (b, 0, 0),
                         memory_space=pltpu.SMEM),
            col, col, col, col, row, row, row, row,
        ],
        out_specs=pl.BlockSpec((1, 1, 128), lambda b, rb: (b, 0, 0)),
        out_shape=jax.ShapeDtypeStruct((B, 1, 128), jnp.float32),
        compiler_params=pltpu.CompilerParams(
            dimension_semantics=("arbitrary", "arbitrary")),
        interpret=interpret,
    )(q, *args)
    return out.reshape(B, 128)


def _photo_body(sl_ref, sr_ref, kxl, kyl, kxr, kyr, sc_ref, out_ref):
    b = pl.program_id(0)
    sc = sc_ref[...]

    def patches(s_ref, kx, ky):
        wx = kx - jnp.floor(kx)
        wy = ky - jnp.floor(ky)
        v = s_ref[0:56, :] * (1.0 - wy) + s_ref[8:64, :] * wy
        v3 = v.reshape(7, 8, v.shape[-1])
        p = v3[:, :7, :] * (1.0 - wx) + v3[:, 1:, :] * wx
        val = v[27:28, :] * (1.0 - wx) + v[28:29, :] * wx
        return p, val

    pl_p, val_l = patches(sl_ref, kxl[...], kyl[...])
    pr_p, val_r = patches(sr_ref, kxr[...], kyr[...])

    t = jnp.sum(jnp.abs(pl_p - pr_p), axis=1)
    adsum = jnp.sum(t, axis=0, keepdims=True)
    diff = adsum * jnp.float32(1.0 / 49.0)

    isb = (val_l > 0.02).astype(jnp.float32)
    wts = sc * isb
    num1 = jnp.sum(diff * wts)
    den = jnp.sum(wts)
    num2 = jnp.sum(_smooth_l1(val_l, val_r, jnp.float32(1.0)) * wts)

    ydiff = jnp.abs(kyl[...] - kyr[...])
    e_num = jnp.sum(ydiff * sc)
    e_den = jnp.sum(sc)
    e_sum = jnp.sum(ydiff)
    nfin = jnp.sum((sc > 0.1).astype(jnp.float32))

    lane = lax.broadcasted_iota(jnp.int32, (1, 128), 1)
    vec = (jnp.where(lane == 0, num1, 0.0) + jnp.where(lane == 1, den, 0.0)
           + jnp.where(lane == 2, num2, 0.0) + jnp.where(lane == 3, e_num, 0.0)
           + jnp.where(lane == 4, e_den, 0.0) + jnp.where(lane == 5, e_sum, 0.0)
           + jnp.where(lane == 6, nfin, 0.0))

    @pl.when(b == 0)
    def _():
        out_ref[...] = jnp.zeros_like(out_ref)

    out_ref[...] += vec


def _photo_call(sl, sr, kxl, kyl, kxr, kyr, sc, interpret=False):
    simg = pl.BlockSpec((64, NP), lambda b: (0, b))
    vec = pl.BlockSpec((1, NP), lambda b: (0, b))
    return pl.pallas_call(
        _photo_body,
        grid=(B,),
        in_specs=[simg, simg, vec, vec, vec, vec, vec],
        out_specs=pl.BlockSpec((1, 128), lambda b: (0, 0)),
        out_shape=jax.ShapeDtypeStruct((1, 128), jnp.float32),
        compiler_params=pltpu.CompilerParams(
            dimension_semantics=("arbitrary",)),
        interpret=interpret,
    )(sl, sr, kxl, kyl, kxr, kyr, sc)


def kernel(lg, rg, kpl, kpr, scores, Q):
    f32 = jnp.float32
    pad = ((0, 0), (0, NP - N))
    klx = jnp.pad(kpl[..., 0], pad, constant_values=100.0)
    kly = jnp.pad(kpl[..., 1], pad, constant_values=100.0)
    krx = jnp.pad(kpr[..., 0], pad, constant_values=100.0)
    kry = jnp.pad(kpr[..., 1], pad, constant_values=100.0)
    sc = jnp.pad(scores, pad, constant_values=0.0)

    imgs = jnp.concatenate([lg.reshape(-1), rg.reshape(-1)])

    sl_g, sr_g = _sc_gather(imgs, klx.reshape(-1), kly.reshape(-1),
                            krx.reshape(-1), kry.reshape(-1))
    sl = sl_g.reshape(GPI, 64, GRP).transpose(1, 0, 2).reshape(64, NKP)
    sr = sr_g.reshape(GPI, 64, GRP).transpose(1, 0, 2).reshape(64, NKP)

    pinn = _cdist_call(Q, klx, kly, krx, sc)

    photo = _photo_call(sl, sr,
                        klx.reshape(1, -1), kly.reshape(1, -1),
                        krx.reshape(1, -1), kry.reshape(1, -1),
                        sc.reshape(1, -1))

    a = photo[0]
    num1, den_w, num2 = a[0], a[1], a[2]
    e_num, e_den, e_sum, nfin = a[3], a[4], a[5], a[6]
    l_epi = jnp.where(e_den > 1e-4, e_num / jnp.maximum(e_den, 1e-12),
                      e_sum / f32(B * N))
    l_masked = jnp.where(den_w < 1e-4, 0.0, num1 / jnp.maximum(den_w, 1e-12))
    l_int = jnp.where(den_w < 1e-4, 0.0, num2 / jnp.maximum(den_w, 1e-12))
    l_photo = l_masked + l_int

    s_sm, s_pen, s_h, cnt = pinn[:, 0], pinn[:, 1], pinn[:, 2], pinn[:, 3]
    msum = jnp.maximum(cnt, 1.0)
    ok = (cnt >= 10.0).astype(f32)
    ls = jnp.sum(ok * s_sm / msum)
    lsl = jnp.sum(ok * s_pen / (msum * K))
    lz = jnp.sum(ok * jnp.abs(s_h / msum))
    vb = jnp.sum(ok)
    vbs = jnp.maximum(vb, 1.0)
    gate = (vb > 0.0).astype(f32)
    l_sm = gate * ls / vbs
    l_sl = gate * lsl / vbs
    l_zm = gate * lz / vbs

    g = (nfin >= 10.0).astype(f32)
    return (l_photo, l_epi, g * l_sm, g * l_sl, g * l_zm)
